# Initial kernel scaffold; baseline (speedup 1.0000x reference)
import jax
import jax.numpy as jnp
from jax.experimental import pallas as pl


def kernel(edge_index, x, w, W1, Wfc, ln_g, ln_b, W2, W3):
    raise NotImplementedError("write your pallas kernel here")


# speedup vs baseline: 1.6215x; 1.6215x over previous
import jax
import jax.numpy as jnp
from jax import lax
from jax.experimental import pallas as pl
from jax.experimental.pallas import tpu as pltpu
from jax.experimental.pallas import tpu_sc as plsc

N = 10000
E = 320000
IN_DIM = 128
D = 64
NC = 2
NS = 16
NT = NS
C = 128
NCH = 158
EP = NT * NCH * C
N2 = 10240
NR = N2 // 2
RPT = NR // NS
DPT = N2 // NS


def _f32(*shape):
    return jax.ShapeDtypeStruct(shape, jnp.float32)


def _sc_kernel(out_type, scratch_types):
    def deco(body):
        built = []

        def call(*args):
            if not built:
                mesh = plsc.VectorSubcoreMesh(
                    core_axis_name="c", subcore_axis_name="s",
                    num_cores=1, num_subcores=NS)
                built.append(pl.kernel(body, out_type=out_type, mesh=mesh,
                                       scratch_types=scratch_types))
            return built[0](*args)

        return call
    return deco


@_sc_kernel(
    out_type=_f32(NR, 128),
    scratch_types=[
        pltpu.VMEM((NCH, C), jnp.int32),
        pltpu.VMEM((NCH, C), jnp.int32),
        pltpu.VMEM((1, C), jnp.int32),
        pltpu.VMEM((1, C), jnp.int32),
        pltpu.VMEM((NCH, C), jnp.float32),
        pltpu.VMEM((C, 128), jnp.float32),
        pltpu.VMEM((64, 128), jnp.float32),
        pltpu.VMEM_SHARED((NR, 128), jnp.float32),
    ],
)
def _sc_scatter(h_hbm, src_hbm, dst_hbm, ew_hbm, zeros_hbm, out_hbm,
                src_v, dst_v, srow_v, drow_v, ew_v, rows_v, zeros_v, agg_sh):
    s = lax.axis_index("s")
    t = s
    pltpu.sync_copy(zeros_hbm, zeros_v)
    for b in range(RPT // 64):
        pltpu.sync_copy(zeros_v, agg_sh.at[pl.ds(s * RPT + b * 64, 64)])
    pltpu.sync_copy(src_hbm.at[t], src_v)
    pltpu.sync_copy(dst_hbm.at[t], dst_v)
    pltpu.sync_copy(ew_hbm.at[t], ew_v)
    plsc.subcore_barrier()

    zeros16 = jnp.zeros((16,), jnp.float32)

    def chunk(j, carry):
        for g in range(C // 16):
            sl16 = pl.ds(g * 16, 16)
            srow_v[0, sl16] = src_v[j, sl16] >> 1
            drow_v[0, sl16] = dst_v[j, sl16] >> 1
        pltpu.sync_copy(h_hbm.at[srow_v.at[0]], rows_v)

        def scale(g, cc):
            sl16 = pl.ds(g * 16, 16)
            wv = ew_v[j, sl16]
            sbv = src_v[j, sl16]
            dbv = dst_v[j, sl16]
            for l in range(16):
                e = g * 16 + l
                wgt = wv[l]
                sb = (sbv[l] & 1) > 0
                db = (dbv[l] & 1) > 0
                for q in range(4):
                    lo_sl = pl.ds(q * 16, 16)
                    hi_sl = pl.ds(64 + q * 16, 16)
                    lo = rows_v[e, lo_sl]
                    hi = rows_v[e, hi_sl]
                    v = jnp.where(sb, hi, lo) * wgt
                    rows_v[e, lo_sl] = jnp.where(db, zeros16, v)
                    rows_v[e, hi_sl] = jnp.where(db, v, zeros16)
            return cc

        lax.fori_loop(0, C // 16, scale, 0)
        pltpu.sync_copy(rows_v, agg_sh.at[drow_v.at[0]], add=True)
        return carry

    lax.fori_loop(0, NCH, chunk, 0)
    plsc.subcore_barrier()
    pltpu.sync_copy(agg_sh.at[pl.ds(s * RPT, RPT)],
                    out_hbm.at[pl.ds(s * RPT, RPT)])


def _tc_ewb(w_pad):
    def body(w_ref, o_ref):
        o_ref[...] = jnp.dot(w_ref[...] * 0.25,
                             jnp.ones((4, 1), jnp.float32),
                             preferred_element_type=jnp.float32)

    return pl.pallas_call(
        body,
        grid=(NCH,),
        in_specs=[pl.BlockSpec((EP // NCH, 4), lambda i: (i, 0))],
        out_specs=pl.BlockSpec((EP // NCH, 1), lambda i: (i, 0)),
        out_shape=_f32(EP, 1),
    )(w_pad)


def _tc_dense1(xp, W1, Wfc, g2, b2, ps, pd):
    def body(x_ref, w1_ref, wfc_ref, g_ref, b_ref, ps_ref, pd_ref,
             h1_ref, f1_ref, ns_ref, nd_ref):
        psb = ps_ref[...]
        pdb = pd_ref[...]
        ns = lax.rsqrt(psb[0, :, 0:1] + 1.0)
        nd = lax.rsqrt(pdb[0, :, 0:1] + 1.0)
        ns_ref[...] = ns
        nd_ref[...] = nd
        xb = x_ref[...]
        h1_ref[...] = ns * jnp.dot(xb, w1_ref[...],
                                   preferred_element_type=jnp.float32)
        f = jnp.dot(xb, wfc_ref[...], preferred_element_type=jnp.float32)
        mu = jnp.mean(f, axis=-1, keepdims=True)
        var = jnp.mean((f - mu) ** 2, axis=-1, keepdims=True)
        fn = g_ref[...] * (f - mu) * lax.rsqrt(var + 1e-5) + b_ref[...]
        f1_ref[...] = jnp.maximum(fn, 0.0)

    return pl.pallas_call(
        body,
        grid=(N2 // 128,),
        in_specs=[
            pl.BlockSpec((128, IN_DIM), lambda i: (i, 0)),
            pl.BlockSpec((IN_DIM, D), lambda i: (0, 0)),
            pl.BlockSpec((IN_DIM, D), lambda i: (0, 0)),
            pl.BlockSpec((1, D), lambda i: (0, 0)),
            pl.BlockSpec((1, D), lambda i: (0, 0)),
            pl.BlockSpec((1, 128, D), lambda i: (0, i, 0)),
            pl.BlockSpec((1, 128, D), lambda i: (0, i, 0)),
        ],
        out_specs=[
            pl.BlockSpec((128, D), lambda i: (i, 0)),
            pl.BlockSpec((128, D), lambda i: (i, 0)),
            pl.BlockSpec((128, 1), lambda i: (i, 0)),
            pl.BlockSpec((128, 1), lambda i: (i, 0)),
        ],
        out_shape=[_f32(N2, D), _f32(N2, D), _f32(N2, 1), _f32(N2, 1)],
    )(xp, W1, Wfc, g2, b2, ps, pd)


def _tc_layer2(agg, h1s, ns2, nd2, f1, W2):
    def body(a_ref, h_ref, ns_ref, nd_ref, f_ref, w_ref, h2_ref):
        a = a_ref[...]
        x1 = jnp.maximum(nd_ref[...] * (a[0] + h_ref[...]), 0.0)
        wb = w_ref[...]
        h2_ref[...] = ns_ref[...] * (
            jnp.dot(x1, wb[:D], preferred_element_type=jnp.float32)
            + jnp.dot(f_ref[...], wb[D:], preferred_element_type=jnp.float32))

    return pl.pallas_call(
        body,
        grid=(N2 // 128,),
        in_specs=[
            pl.BlockSpec((1, 128, D), lambda i: (0, i, 0)),
            pl.BlockSpec((128, D), lambda i: (i, 0)),
            pl.BlockSpec((128, 1), lambda i: (i, 0)),
            pl.BlockSpec((128, 1), lambda i: (i, 0)),
            pl.BlockSpec((128, D), lambda i: (i, 0)),
            pl.BlockSpec((2 * D, D), lambda i: (0, 0)),
        ],
        out_specs=pl.BlockSpec((128, D), lambda i: (i, 0)),
        out_shape=_f32(N2, D),
    )(agg, h1s, ns2, nd2, f1, W2)


def _tc_layer3(agg, h2s, ns2, nd2, W3):
    def body(a_ref, h_ref, ns_ref, nd_ref, w_ref, h3_ref):
        a = a_ref[...]
        x2 = jnp.maximum(nd_ref[...] * (a[0] + h_ref[...]), 0.0)
        h3_ref[...] = ns_ref[...] * jnp.dot(
            x2, w_ref[...], preferred_element_type=jnp.float32)

    return pl.pallas_call(
        body,
        grid=(N2 // 128,),
        in_specs=[
            pl.BlockSpec((1, 128, D), lambda i: (0, i, 0)),
            pl.BlockSpec((128, D), lambda i: (i, 0)),
            pl.BlockSpec((128, 1), lambda i: (i, 0)),
            pl.BlockSpec((128, 1), lambda i: (i, 0)),
            pl.BlockSpec((D, D), lambda i: (0, 0)),
        ],
        out_specs=pl.BlockSpec((128, D), lambda i: (i, 0)),
        out_shape=_f32(N2, D),
    )(agg, h2s, ns2, nd2, W3)


def _tc_final(agg, h3s, nd2):
    def body(a_ref, h_ref, nd_ref, o_ref):
        i = pl.program_id(0)
        a = a_ref[...]
        x3 = jnp.maximum(nd_ref[...] * (a[0] + h_ref[...]), 0.0)
        rows = i * 128 + lax.broadcasted_iota(jnp.int32, (128, 1), 0)
        x3 = jnp.where(rows < N, x3, 0.0)
        bs = jnp.sum(x3, axis=0, keepdims=True)

        @pl.when(i == 0)
        def _():
            o_ref[...] = jnp.zeros_like(o_ref)

        o_ref[...] += bs

    return pl.pallas_call(
        body,
        grid=(N2 // 128,),
        in_specs=[
            pl.BlockSpec((1, 128, D), lambda i: (0, i, 0)),
            pl.BlockSpec((128, D), lambda i: (i, 0)),
            pl.BlockSpec((128, 1), lambda i: (i, 0)),
        ],
        out_specs=pl.BlockSpec((1, D), lambda i: (0, 0)),
        out_shape=_f32(1, D),
    )(agg, h3s, nd2)


def kernel(edge_index, x, w, W1, Wfc, ln_g, ln_b, W2, W3):
    npad = EP - E
    pad_idx = N + (jnp.arange(npad, dtype=jnp.int32) % (N2 - N))
    src = jnp.concatenate([edge_index[0], pad_idx]).reshape(NT, NCH, C)
    dst = jnp.concatenate([edge_index[1], pad_idx]).reshape(NT, NCH, C)
    w_pad = jnp.pad(w, ((0, npad), (0, 0)))
    xp = jnp.pad(x, ((0, N2 - N), (0, 0)))
    g2 = ln_g.reshape(1, D)
    b2 = ln_b.reshape(1, D)
    zeros_r = jnp.zeros((64, 128), jnp.float32)
    one0 = jnp.concatenate([jnp.ones((E,), jnp.float32),
                            jnp.zeros((EP - E,), jnp.float32)])
    one0 = one0.reshape(NT, NCH, C)

    ones_h2 = jnp.ones((NR, 128), jnp.float32)

    ps = _sc_scatter(ones_h2, src, src, one0, zeros_r).reshape(1, N2, D)
    pd = _sc_scatter(ones_h2, dst, dst, one0, zeros_r).reshape(1, N2, D)
    ewb = _tc_ewb(w_pad).reshape(NT, NCH, C)
    h1s, f1, ns2, nd2 = _tc_dense1(xp, W1, Wfc, g2, b2, ps, pd)

    def layer(hs):
        return _sc_scatter(hs.reshape(NR, 128), src, dst, ewb,
                           zeros_r).reshape(1, N2, D)

    agg1 = layer(h1s)
    h2s = _tc_layer2(agg1, h1s, ns2, nd2, f1, W2)
    agg2 = layer(h2s)
    h3s = _tc_layer3(agg2, h2s, ns2, nd2, W3)
    agg3 = layer(h3s)
    return _tc_final(agg3, h3s, nd2)
